# baseline (speedup 1.0000x reference)
import jax
import jax.numpy as jnp
from jax.experimental import pallas as pl
from jax.experimental.pallas import tpu as pltpu

_CHUNK = 64


def _nn_body(r_ref, qTm2_ref, b2_ref, a2_ref, d_ref, i_ref):
    aTm2 = qTm2_ref[0, 0]
    a2 = a2_ref[0, 0]
    P2 = r_ref.shape[2]
    C = _CHUNK
    W = 1
    acc_v = [None] * W
    acc_c = [None] * W
    for c in range(P2 // C):
        b = r_ref[0, 0, c * C:(c + 1) * C]
        b2 = b2_ref[0, 0, c * C:(c + 1) * C]
        cross2 = jnp.dot(b, aTm2)
        d2c = (a2 + b2) + cross2
        w = c % W
        if acc_v[w] is None:
            acc_v[w] = d2c
            acc_c[w] = jnp.full(d2c.shape, c, jnp.int32)
        else:
            mask = d2c < acc_v[w]
            acc_c[w] = jnp.where(mask, c, acc_c[w])
            acc_v[w] = jnp.minimum(d2c, acc_v[w])
    rowi = jax.lax.broadcasted_iota(jnp.int32, acc_v[0].shape, 0)
    av = jnp.concatenate(acc_v, axis=0)
    aj = jnp.concatenate([acc_c[w] * C + rowi for w in range(W)], axis=0)
    m = jnp.min(av, axis=0, keepdims=True)
    idx = jnp.min(jnp.where(av == m, aj, P2), axis=0, keepdims=True)
    d_ref[0] = m
    i_ref[0] = idx


def kernel(x, y, block_q=256):
    N, P1, D = x.shape
    P2 = y.shape[1]
    nb = P1 // block_q
    refs = jnp.stack([y, x])
    qTm2 = -2.0 * jnp.stack(
        [jnp.swapaxes(x, 1, 2), jnp.swapaxes(y, 1, 2)])
    x2 = jnp.sum(x * x, axis=-1)
    y2 = jnp.sum(y * y, axis=-1)
    b2 = jnp.stack([y2, x2])[..., None]
    a2 = jnp.stack([x2, y2])[:, :, None, :]
    dists, idx = pl.pallas_call(
        _nn_body,
        grid=(2, N, nb),
        in_specs=[
            pl.BlockSpec((1, 1, P2, D), lambda d, n, i: (d, n, 0, 0)),
            pl.BlockSpec((1, 1, D, block_q), lambda d, n, i: (d, n, 0, i)),
            pl.BlockSpec((1, 1, P2, 1), lambda d, n, i: (d, n, 0, 0)),
            pl.BlockSpec((1, 1, 1, block_q), lambda d, n, i: (d, n, 0, i)),
        ],
        out_specs=[
            pl.BlockSpec((1, 1, block_q),
                         lambda d, n, i, nb=nb, N=N: (d * N * nb + n * nb + i, 0, 0)),
            pl.BlockSpec((1, 1, block_q),
                         lambda d, n, i, nb=nb, N=N: (d * N * nb + n * nb + i, 0, 0)),
        ],
        out_shape=[
            jax.ShapeDtypeStruct((2 * N * nb, 1, block_q), jnp.float32),
            jax.ShapeDtypeStruct((2 * N * nb, 1, block_q), jnp.int32),
        ],
        compiler_params=pltpu.CompilerParams(
            dimension_semantics=("parallel", "parallel", "parallel")),
    )(refs, qTm2, b2, a2)
    dists = dists.reshape(2, N, P1)
    idx = idx.reshape(2, N, P1)
    return dists[0], dists[1], idx[0], idx[1]


# speedup vs baseline: 1.1002x; 1.1002x over previous
import jax
import jax.numpy as jnp
from jax.experimental import pallas as pl
from jax.experimental.pallas import tpu as pltpu

_CHUNK = 64


def _nn_body(r_ref, qT_ref, d_ref, i_ref):
    aT = qT_ref[0]
    P2 = r_ref.shape[1]
    aTm2 = -2.0 * aT
    a2 = jnp.sum(aT * aT, axis=0, keepdims=True)
    C = _CHUNK
    acc_v = None
    for c in range(P2 // C):
        b = r_ref[0, c * C:(c + 1) * C]
        b2 = jnp.sum(b * b, axis=1, keepdims=True)
        cross2 = jnp.dot(b, aTm2)
        d2c = (a2 + b2) + cross2
        if c == 0:
            acc_v = d2c
            acc_c = jnp.zeros(d2c.shape, jnp.int32)
        else:
            mask = d2c < acc_v
            acc_v = jnp.where(mask, d2c, acc_v)
            acc_c = jnp.where(mask, c, acc_c)
    m = jnp.min(acc_v, axis=0, keepdims=True)
    rowi = jax.lax.broadcasted_iota(jnp.int32, acc_v.shape, 0)
    j = acc_c * C + rowi
    idx = jnp.min(jnp.where(acc_v == m, j, P2), axis=0, keepdims=True)
    d_ref[0] = m
    i_ref[0] = idx


def _nn_dir(q, r, block_q=256):
    N, P1, D = q.shape
    P2 = r.shape[1]
    nb = P1 // block_q
    qT = jnp.swapaxes(q, 1, 2)
    dists, idx = pl.pallas_call(
        _nn_body,
        grid=(N, nb),
        in_specs=[
            pl.BlockSpec((1, P2, D), lambda n, i: (n, 0, 0)),
            pl.BlockSpec((1, D, block_q), lambda n, i: (n, 0, i)),
        ],
        out_specs=[
            pl.BlockSpec((1, 1, block_q), lambda n, i, nb=nb: (n * nb + i, 0, 0)),
            pl.BlockSpec((1, 1, block_q), lambda n, i, nb=nb: (n * nb + i, 0, 0)),
        ],
        out_shape=[
            jax.ShapeDtypeStruct((N * nb, 1, block_q), jnp.float32),
            jax.ShapeDtypeStruct((N * nb, 1, block_q), jnp.int32),
        ],
        compiler_params=pltpu.CompilerParams(
            dimension_semantics=("parallel", "parallel")),
    )(r, qT)
    return dists.reshape(N, P1), idx.reshape(N, P1)


def kernel(x, y):
    cham_x, idx_x = _nn_dir(x, y)
    cham_y, idx_y = _nn_dir(y, x)
    return cham_x, cham_y, idx_x, idx_y


# speedup vs baseline: 1.1239x; 1.0215x over previous
import jax
import jax.numpy as jnp
from jax.experimental import pallas as pl
from jax.experimental.pallas import tpu as pltpu

_CHUNK = 32


def _nn_body(r_ref, qT_ref, d_ref, i_ref):
    aT = qT_ref[0]
    P2 = r_ref.shape[1]
    aTm2 = -2.0 * aT
    a2 = jnp.sum(aT * aT, axis=0, keepdims=True)
    C = _CHUNK
    acc_v = None
    for c in range(P2 // C):
        b = r_ref[0, c * C:(c + 1) * C]
        b2 = jnp.sum(b * b, axis=1, keepdims=True)
        cross2 = jnp.dot(b, aTm2)
        d2c = (a2 + b2) + cross2
        if c == 0:
            acc_v = d2c
            acc_c = jnp.zeros(d2c.shape, jnp.int32)
        else:
            mask = d2c < acc_v
            acc_v = jnp.where(mask, d2c, acc_v)
            acc_c = jnp.where(mask, c, acc_c)
    m = jnp.min(acc_v, axis=0, keepdims=True)
    rowi = jax.lax.broadcasted_iota(jnp.int32, acc_v.shape, 0)
    j = acc_c * C + rowi
    idx = jnp.min(jnp.where(acc_v == m, j, P2), axis=0, keepdims=True)
    d_ref[0] = m
    i_ref[0] = idx


def _nn_dir(q, r, block_q=256):
    N, P1, D = q.shape
    P2 = r.shape[1]
    nb = P1 // block_q
    qT = jnp.swapaxes(q, 1, 2)
    dists, idx = pl.pallas_call(
        _nn_body,
        grid=(N, nb),
        in_specs=[
            pl.BlockSpec((1, P2, D), lambda n, i: (n, 0, 0)),
            pl.BlockSpec((1, D, block_q), lambda n, i: (n, 0, i)),
        ],
        out_specs=[
            pl.BlockSpec((1, 1, block_q), lambda n, i, nb=nb: (n * nb + i, 0, 0)),
            pl.BlockSpec((1, 1, block_q), lambda n, i, nb=nb: (n * nb + i, 0, 0)),
        ],
        out_shape=[
            jax.ShapeDtypeStruct((N * nb, 1, block_q), jnp.float32),
            jax.ShapeDtypeStruct((N * nb, 1, block_q), jnp.int32),
        ],
        compiler_params=pltpu.CompilerParams(
            dimension_semantics=("parallel", "parallel")),
    )(r, qT)
    return dists.reshape(N, P1), idx.reshape(N, P1)


def kernel(x, y):
    cham_x, idx_x = _nn_dir(x, y)
    cham_y, idx_y = _nn_dir(y, x)
    return cham_x, cham_y, idx_x, idx_y
